# baseline (speedup 1.0000x reference)
import functools

import jax
import jax.numpy as jnp
from jax import lax
from jax.experimental import pallas as pl
from jax.experimental.pallas import tpu as pltpu
from jax.experimental.pallas import tpu_sc as plsc

B = 16384
UD = 64
ZD = 32
BT = 2048
OH = 128

_NC = 2
_NS = 16
_NW = _NC * _NS
_BPW = B // _NW
_CHUNK = 128
_NCHUNK = _BPW // _CHUNK


def _sc_gather_body(uid_hbm, zid_hbm, utab_hbm, ztab_hbm, uout_hbm, zout_hbm,
                    uidx_v, urows_v, zidx_v, zrows_v, usem, zsem):
    wid = lax.axis_index("s") * _NC + lax.axis_index("c")
    base = wid * _BPW
    pltpu.sync_copy(uid_hbm.at[pl.ds(wid * _NCHUNK, _NCHUNK)], uidx_v)
    pltpu.sync_copy(zid_hbm.at[pl.ds(wid * _NCHUNK, _NCHUNK)], zidx_v)
    copies = []
    for j in range(_NCHUNK):
        copies.append(pltpu.async_copy(
            utab_hbm.at[uidx_v.at[j]],
            urows_v.at[pl.ds(j * _CHUNK, _CHUNK)], usem))
        copies.append(pltpu.async_copy(
            ztab_hbm.at[zidx_v.at[j]],
            zrows_v.at[pl.ds(j * _CHUNK, _CHUNK)], zsem))
    for c in copies:
        c.wait()
    pltpu.sync_copy(urows_v, uout_hbm.at[pl.ds(base, _BPW)])
    pltpu.sync_copy(zrows_v, zout_hbm.at[pl.ds(base, _BPW)])


@functools.cache
def _sc_gather():
    return functools.partial(
        pl.kernel,
        mesh=plsc.VectorSubcoreMesh(core_axis_name="c", subcore_axis_name="s"),
        compiler_params=pltpu.CompilerParams(use_tc_tiling_on_sc=False),
        out_type=[jax.ShapeDtypeStruct((B, UD), jnp.float32),
                  jax.ShapeDtypeStruct((B, ZD), jnp.float32)],
        scratch_types=[pltpu.VMEM((_NCHUNK, _CHUNK), jnp.int32),
                       pltpu.VMEM((_BPW, UD), jnp.float32),
                       pltpu.VMEM((_NCHUNK, _CHUNK), jnp.int32),
                       pltpu.VMEM((_BPW, ZD), jnp.float32),
                       pltpu.SemaphoreType.DMA,
                       pltpu.SemaphoreType.DMA],
    )(_sc_gather_body)


def _tc_body(idx_ref, u_ref, z_ref, w1u_ref, w1z_ref, ws_ref, b1_ref,
             w2_ref, b2_ref, o_ref):
    g = idx_ref[:, 0:1]
    occ = idx_ref[:, 1:2]
    age = idx_ref[:, 2:3]
    dow = idx_ref[:, 3:4]
    hod = idx_ref[:, 4:5]
    cross = lax.rem(dow * 24 + hod, 34)
    cols = lax.broadcasted_iota(jnp.int32, (BT, OH), 1)
    oh = ((cols == g)
          | (cols == occ + 2)
          | (cols == age + 24)
          | (cols == dow + 31)
          | (cols == hod + 38)
          | (cols == cross + 62)).astype(jnp.float32)
    h1 = (jnp.dot(u_ref[...], w1u_ref[...], preferred_element_type=jnp.float32)
          + jnp.dot(z_ref[...], w1z_ref[...], preferred_element_type=jnp.float32)
          + jnp.dot(oh, ws_ref[...], preferred_element_type=jnp.float32)
          + b1_ref[...])
    h1 = jnp.maximum(h1, 0.0)
    h2 = jnp.dot(h1, w2_ref[...], preferred_element_type=jnp.float32) + b2_ref[...]
    h2 = jnp.maximum(h2, 0.0)
    ssq = jnp.sum(h2 * h2, axis=1, keepdims=True)
    o_ref[...] = h2 * lax.rsqrt(jnp.maximum(ssq, 1e-12))


def _tc_tower(idx_packed, user_rows, zip_rows, w1u, w1z, wsmall, b1r, w2, b2r,
              interpret=False):
    return pl.pallas_call(
        _tc_body,
        grid=(B // BT,),
        in_specs=[
            pl.BlockSpec((BT, 8), lambda i: (i, 0)),
            pl.BlockSpec((BT, UD), lambda i: (i, 0)),
            pl.BlockSpec((BT, ZD), lambda i: (i, 0)),
            pl.BlockSpec((UD, 64), lambda i: (0, 0)),
            pl.BlockSpec((ZD, 64), lambda i: (0, 0)),
            pl.BlockSpec((OH, 64), lambda i: (0, 0)),
            pl.BlockSpec((1, 64), lambda i: (0, 0)),
            pl.BlockSpec((64, 32), lambda i: (0, 0)),
            pl.BlockSpec((1, 32), lambda i: (0, 0)),
        ],
        out_specs=pl.BlockSpec((BT, 32), lambda i: (i, 0)),
        out_shape=jax.ShapeDtypeStruct((B, 32), jnp.float32),
        interpret=interpret,
    )(idx_packed, user_rows, zip_rows, w1u, w1z, wsmall, b1r, w2, b2r)


def kernel(user_gender, user_id, user_occupation_label, user_zip_code,
           bucketized_user_age, day_of_week, hour_of_day, user_table,
           zip_table, cross_table, W1, b1, W2, b2):
    uid = user_id.astype(jnp.int32).reshape(B // _CHUNK, _CHUNK)
    zid = user_zip_code.astype(jnp.int32).reshape(B // _CHUNK, _CHUNK)
    user_rows, zip_rows = _sc_gather()(uid, zid, user_table, zip_table)

    g = user_gender.astype(jnp.int32)
    zeros = jnp.zeros_like(g)
    idx_packed = jnp.stack(
        [g, user_occupation_label.astype(jnp.int32),
         bucketized_user_age.astype(jnp.int32),
         day_of_week.astype(jnp.int32), hour_of_day.astype(jnp.int32),
         zeros, zeros, zeros], axis=1)

    c2 = cross_table @ W1[158:174]
    wsmall = jnp.concatenate(
        [W1[0:2], W1[66:88], W1[120:158], c2,
         jnp.zeros((OH - 97, 64), jnp.float32)], axis=0)
    return _tc_tower(idx_packed, user_rows, zip_rows,
                     W1[2:66], W1[88:120], wsmall,
                     b1.reshape(1, 64), W2.reshape(64, 32), b2.reshape(1, 32))
